# Initial kernel scaffold; baseline (speedup 1.0000x reference)
import jax
import jax.numpy as jnp
from jax.experimental import pallas as pl


def kernel(atom_types, bond_types, edge_index, atom_table, bond_table, W1, b1, W2, b2, W3, b3):
    raise NotImplementedError("write your pallas kernel here")


# speedup vs baseline: 3.0937x; 3.0937x over previous
import functools
import jax
import jax.numpy as jnp
from jax import lax
from jax.experimental import pallas as pl
from jax.experimental.pallas import tpu as pltpu

BN = 2000


def _k1_body(z1, y0, dinv, W1, b1, y1):
    a = dinv[...] * (z1[...] + y0[...])
    h = jnp.dot(a, W1[...], preferred_element_type=jnp.float32) + b1[...]
    y1[...] = dinv[...] * jnp.maximum(h, 0.0)


def _layer1(z1, y0, dinv2d, W1, b1, n):
    nb = n // BN
    return pl.pallas_call(
        _k1_body,
        grid=(nb,),
        in_specs=[
            pl.BlockSpec((BN, 64), lambda i: (i, 0)),
            pl.BlockSpec((BN, 64), lambda i: (i, 0)),
            pl.BlockSpec((BN, 1), lambda i: (i, 0)),
            pl.BlockSpec((64, 128), lambda i: (0, 0)),
            pl.BlockSpec((1, 128), lambda i: (0, 0)),
        ],
        out_specs=pl.BlockSpec((BN, 128), lambda i: (i, 0)),
        out_shape=jax.ShapeDtypeStruct((n, 128), jnp.float32),
    )(z1, y0, dinv2d, W1, b1.reshape(1, 128))


def _k2_body(z2, y1, dinv, c, W2, b2, W3, b3, nfloat, out, racc):
    i = pl.program_id(0)
    a = dinv[...] * (z2[...] + y1[...])
    h = jnp.dot(a, W2[...], preferred_element_type=jnp.float32) + b2[...]
    x2 = jnp.maximum(h, 0.0)
    r = lax.dot_general(c[...], x2, (((0,), (0,)), ((), ())),
                        preferred_element_type=jnp.float32)

    @pl.when(i == 0)
    def _():
        racc[...] = r

    @pl.when(i > 0)
    def _():
        racc[...] = racc[...] + r

    @pl.when(i == pl.num_programs(0) - 1)
    def _():
        out[...] = jnp.dot(racc[...] * nfloat[0, 0], W3[...],
                           preferred_element_type=jnp.float32) + b3[...]


def _layer23(z2, y1, dinv2d, c2d, W2, b2, W3, b3, n):
    nb = n // BN
    nfloat = jnp.full((1, 1), 1.0 / n, dtype=jnp.float32)
    return pl.pallas_call(
        _k2_body,
        grid=(nb,),
        in_specs=[
            pl.BlockSpec((BN, 128), lambda i: (i, 0)),
            pl.BlockSpec((BN, 128), lambda i: (i, 0)),
            pl.BlockSpec((BN, 1), lambda i: (i, 0)),
            pl.BlockSpec((BN, 1), lambda i: (i, 0)),
            pl.BlockSpec((128, 256), lambda i: (0, 0)),
            pl.BlockSpec((1, 256), lambda i: (0, 0)),
            pl.BlockSpec((256, 512), lambda i: (0, 0)),
            pl.BlockSpec((1, 512), lambda i: (0, 0)),
            pl.BlockSpec((1, 1), lambda i: (0, 0)),
        ],
        out_specs=pl.BlockSpec((1, 512), lambda i: (0, 0)),
        out_shape=jax.ShapeDtypeStruct((1, 512), jnp.float32),
        scratch_shapes=[pltpu.VMEM((1, 256), jnp.float32)],
    )(z2, y1, dinv2d, c2d, W2, b2.reshape(1, 256), W3, b3.reshape(1, 512),
      nfloat)


def kernel(atom_types, bond_types, edge_index, atom_table, bond_table,
           W1, b1, W2, b2, W3, b3):
    src, dst = edge_index[0], edge_index[1]
    n = atom_types.shape[0]
    ewtab = bond_table.mean(axis=-1)
    ew = ewtab[bond_types]
    deg = jax.ops.segment_sum(ew, dst, num_segments=n) + 1.0
    dinv = jnp.where(deg > 0, deg ** -0.5, 0.0)
    dinv2d = dinv[:, None]
    x0 = atom_table[atom_types]
    y0 = dinv2d * x0
    z1 = jax.ops.segment_sum(y0[src] * ew[:, None], dst, num_segments=n)
    y1 = _layer1(z1, y0, dinv2d, W1, b1, n)
    z2 = jax.ops.segment_sum(y1[src] * ew[:, None], dst, num_segments=n)
    u = jax.ops.segment_sum(ew * dinv[dst], src, num_segments=n)
    c = dinv * u + dinv * dinv
    return _layer23(z2, y1, dinv2d, c[:, None], W2, b2, W3, b3, n)


# speedup vs baseline: 16.2131x; 5.2406x over previous
import functools
import jax
import jax.numpy as jnp
from jax import lax
from jax.experimental import pallas as pl
from jax.experimental.pallas import tpu as pltpu
from jax.experimental.pallas import tpu_sc as plsc

BN = 2000

NC, NS = 2, 16
IW = 128
RPB = 8
EB = RPB * IW
SW = 16


def _sc_edge_scatter(y_slices, src2d, dst2d, ew, zeros16, n):
    S = len(y_slices)
    nblk = src2d.shape[0] // RPB
    per_tile = nblk // NS
    chunk = -(-(n // NS) // 8) * 8
    last = n - (NS - 1) * chunk

    mesh = plsc.VectorSubcoreMesh(core_axis_name="c", subcore_axis_name="s")

    @functools.partial(
        pl.kernel, mesh=mesh,
        out_type=[jax.ShapeDtypeStruct((n, SW), jnp.float32)] * S,
        scratch_types=[
            pltpu.VMEM((RPB, IW), jnp.int32),
            pltpu.VMEM((RPB, IW), jnp.int32),
            pltpu.VMEM((EB,), jnp.float32),
            pltpu.VMEM((EB, SW), jnp.float32),
            pltpu.VMEM_SHARED((n, SW), jnp.float32),
            pltpu.SemaphoreType.DMA,
        ],
        compiler_params=pltpu.CompilerParams(use_tc_tiling_on_sc=False),
    )
    def scat(*refs):
        ys = refs[:S]
        src_h, dst_h, ew_h, zer_h = refs[S:S + 4]
        zs = refs[S + 4:2 * S + 4]
        srcv, dstv, ewv, rows, zsh, sem = refs[2 * S + 4:]
        cid = lax.axis_index("c")
        sid = lax.axis_index("s")

        def chunk_copy(mk_src, mk_dst):
            @pl.when(sid < NS - 1)
            def _():
                sl = pl.ds(sid * chunk, chunk)
                pltpu.sync_copy(mk_src(sl), mk_dst(sl))

            @pl.when(sid == NS - 1)
            def _():
                sl = pl.ds((NS - 1) * chunk, last)
                pltpu.sync_copy(mk_src(sl), mk_dst(sl))

        def do_slice(y_h, z_h):
            chunk_copy(lambda sl: zer_h.at[sl, :], lambda sl: zsh.at[sl, :])
            plsc.subcore_barrier()

            def block_body(i, _):
                bid = sid + NS * i
                pltpu.sync_copy(src_h.at[pl.ds(bid * RPB, RPB), :], srcv)
                pltpu.sync_copy(dst_h.at[pl.ds(bid * RPB, RPB), :], dstv)
                pltpu.sync_copy(ew_h.at[pl.ds(bid * EB, EB)], ewv)
                cps = [
                    pltpu.async_copy(
                        y_h.at[srcv.at[j]],
                        rows.at[pl.ds(j * IW, IW), :], sem)
                    for j in range(RPB)
                ]
                for cp in cps:
                    cp.wait()

                def scale_body(g, _):
                    wv = ewv[pl.ds(g * 16, 16)]
                    for l in range(16):
                        e = g * 16 + l
                        rows[e, pl.ds(0, SW)] = rows[e, pl.ds(0, SW)] * wv[l]
                    return 0

                lax.fori_loop(0, EB // 16, scale_body, 0)
                for j in range(RPB):
                    pltpu.sync_copy(rows.at[pl.ds(j * IW, IW), :],
                                    zsh.at[dstv.at[j]], add=True)
                return 0

            lax.fori_loop(0, per_tile, block_body, 0)
            plsc.subcore_barrier()
            chunk_copy(lambda sl: zsh.at[sl, :], lambda sl: z_h.at[sl, :])
            plsc.subcore_barrier()

        for kk in range(S // 2):
            for cc in range(NC):
                @pl.when(cid == cc)
                def _(kk=kk, cc=cc):
                    do_slice(ys[NC * kk + cc], zs[NC * kk + cc])

    return scat(*y_slices, src2d, dst2d, ew, zeros16)


def _sc_deg_ew(bt2d, dst2d, ewtab16, zeros1, n):
    epad = bt2d.shape[0] * IW
    nblk = bt2d.shape[0] // RPB
    per_tile = -(-nblk // (NS * NC))
    chunk = -(-(n // NS) // 8) * 8
    last = n - (NS - 1) * chunk

    mesh = plsc.VectorSubcoreMesh(core_axis_name="c", subcore_axis_name="s")

    @functools.partial(
        pl.kernel, mesh=mesh,
        out_type=[jax.ShapeDtypeStruct((epad,), jnp.float32),
                  jax.ShapeDtypeStruct((NC, n), jnp.float32)],
        scratch_types=[
            pltpu.VMEM((RPB, IW), jnp.int32),
            pltpu.VMEM((RPB, IW), jnp.int32),
            pltpu.VMEM((EB,), jnp.float32),
            pltpu.VMEM((16,), jnp.float32),
            pltpu.VMEM_SHARED((n,), jnp.float32),
            pltpu.SemaphoreType.DMA,
        ],
        compiler_params=pltpu.CompilerParams(use_tc_tiling_on_sc=False,
                                             needs_layout_passes=False),
    )
    def degk(bt_h, dst_h, tab_h, zer_h, ew_h, deg_h, btv, dstv, ewv, tabv,
             dsh, sem):
        cid = lax.axis_index("c")
        sid = lax.axis_index("s")
        wid = sid * NC + cid
        pltpu.sync_copy(tab_h, tabv)

        @pl.when(sid < NS - 1)
        def _():
            pltpu.sync_copy(zer_h.at[pl.ds(sid * chunk, chunk)],
                            dsh.at[pl.ds(sid * chunk, chunk)])

        @pl.when(sid == NS - 1)
        def _():
            pltpu.sync_copy(zer_h.at[pl.ds((NS - 1) * chunk, last)],
                            dsh.at[pl.ds((NS - 1) * chunk, last)])

        plsc.subcore_barrier()

        def block_body(i, _):
            bid = wid + NS * NC * i

            @pl.when(bid < nblk)
            def _():
                pltpu.sync_copy(bt_h.at[pl.ds(bid * RPB, RPB), :], btv)
                pltpu.sync_copy(dst_h.at[pl.ds(bid * RPB, RPB), :], dstv)

                def gat(g, _):
                    j, k = g // (IW // 16), g % (IW // 16)
                    bt16 = btv[j, pl.ds(k * 16, 16)]
                    ewv[pl.ds(g * 16, 16)] = plsc.load_gather(tabv, [bt16])
                    return 0

                lax.fori_loop(0, EB // 16, gat, 0)
                pltpu.sync_copy(ewv, ew_h.at[pl.ds(bid * EB, EB)])
                for j in range(RPB):
                    pltpu.sync_copy(ewv.at[pl.ds(j * IW, IW)],
                                    dsh.at[dstv.at[j]], add=True)
            return 0

        lax.fori_loop(0, per_tile, block_body, 0)
        plsc.subcore_barrier()

        @pl.when(sid == 0)
        def _():
            pltpu.sync_copy(dsh, deg_h.at[cid])

    return degk(bt2d, dst2d, ewtab16, zeros1)


def _sc_u(src2d, dst2d, ew, dinv, zeros1, n):
    nblk = src2d.shape[0] // RPB
    per_tile = -(-nblk // (NS * NC))
    chunk = -(-(n // NS) // 8) * 8
    last = n - (NS - 1) * chunk

    mesh = plsc.VectorSubcoreMesh(core_axis_name="c", subcore_axis_name="s")

    @functools.partial(
        pl.kernel, mesh=mesh,
        out_type=jax.ShapeDtypeStruct((NC, n), jnp.float32),
        scratch_types=[
            pltpu.VMEM((RPB, IW), jnp.int32),
            pltpu.VMEM((RPB, IW), jnp.int32),
            pltpu.VMEM((EB,), jnp.float32),
            pltpu.VMEM((EB,), jnp.float32),
            pltpu.VMEM((n,), jnp.float32),
            pltpu.VMEM_SHARED((n,), jnp.float32),
            pltpu.SemaphoreType.DMA,
        ],
        compiler_params=pltpu.CompilerParams(use_tc_tiling_on_sc=False,
                                             needs_layout_passes=False),
    )
    def uk(src_h, dst_h, ew_h, dinv_h, zer_h, u_h, srcv, dstv, ewv, valv,
           dinvv, ush, sem):
        cid = lax.axis_index("c")
        sid = lax.axis_index("s")
        wid = sid * NC + cid
        pltpu.sync_copy(dinv_h, dinvv)

        @pl.when(sid < NS - 1)
        def _():
            pltpu.sync_copy(zer_h.at[pl.ds(sid * chunk, chunk)],
                            ush.at[pl.ds(sid * chunk, chunk)])

        @pl.when(sid == NS - 1)
        def _():
            pltpu.sync_copy(zer_h.at[pl.ds((NS - 1) * chunk, last)],
                            ush.at[pl.ds((NS - 1) * chunk, last)])

        plsc.subcore_barrier()

        def block_body(i, _):
            bid = wid + NS * NC * i

            @pl.when(bid < nblk)
            def _():
                pltpu.sync_copy(src_h.at[pl.ds(bid * RPB, RPB), :], srcv)
                pltpu.sync_copy(dst_h.at[pl.ds(bid * RPB, RPB), :], dstv)
                pltpu.sync_copy(ew_h.at[pl.ds(bid * EB, EB)], ewv)

                def gat(g, _):
                    j, k = g // (IW // 16), g % (IW // 16)
                    d16 = dstv[j, pl.ds(k * 16, 16)]
                    dv = plsc.load_gather(dinvv, [d16])
                    valv[pl.ds(g * 16, 16)] = dv * ewv[pl.ds(g * 16, 16)]
                    return 0

                lax.fori_loop(0, EB // 16, gat, 0)
                for j in range(RPB):
                    pltpu.sync_copy(valv.at[pl.ds(j * IW, IW)],
                                    ush.at[srcv.at[j]], add=True)
            return 0

        lax.fori_loop(0, per_tile, block_body, 0)
        plsc.subcore_barrier()

        @pl.when(sid == 0)
        def _():
            pltpu.sync_copy(ush, u_h.at[cid])

    return uk(src2d, dst2d, ew, dinv, zeros1)


def _sc_aggregate(y, src2d, dst2d, ew, zeros16, n, d):
    ys = tuple(y[:, SW * s:SW * (s + 1)] for s in range(d // SW))
    zs = _sc_edge_scatter(ys, src2d, dst2d, ew, zeros16, n)
    return jnp.concatenate(zs, axis=1)


def _k1_body(z1, y0, dinv, W1, b1, y1):
    a = dinv[...] * (z1[...] + y0[...])
    h = jnp.dot(a, W1[...], preferred_element_type=jnp.float32) + b1[...]
    y1[...] = dinv[...] * jnp.maximum(h, 0.0)


def _layer1(z1, y0, dinv2d, W1, b1, n):
    nb = n // BN
    return pl.pallas_call(
        _k1_body,
        grid=(nb,),
        in_specs=[
            pl.BlockSpec((BN, 64), lambda i: (i, 0)),
            pl.BlockSpec((BN, 64), lambda i: (i, 0)),
            pl.BlockSpec((BN, 1), lambda i: (i, 0)),
            pl.BlockSpec((64, 128), lambda i: (0, 0)),
            pl.BlockSpec((1, 128), lambda i: (0, 0)),
        ],
        out_specs=pl.BlockSpec((BN, 128), lambda i: (i, 0)),
        out_shape=jax.ShapeDtypeStruct((n, 128), jnp.float32),
    )(z1, y0, dinv2d, W1, b1.reshape(1, 128))


def _k2_body(z2, y1, dinv, c, W2, b2, W3, b3, nfloat, out, racc):
    i = pl.program_id(0)
    a = dinv[...] * (z2[...] + y1[...])
    h = jnp.dot(a, W2[...], preferred_element_type=jnp.float32) + b2[...]
    x2 = jnp.maximum(h, 0.0)
    r = lax.dot_general(c[...], x2, (((0,), (0,)), ((), ())),
                        preferred_element_type=jnp.float32)

    @pl.when(i == 0)
    def _():
        racc[...] = r

    @pl.when(i > 0)
    def _():
        racc[...] = racc[...] + r

    @pl.when(i == pl.num_programs(0) - 1)
    def _():
        out[...] = jnp.dot(racc[...] * nfloat[0, 0], W3[...],
                           preferred_element_type=jnp.float32) + b3[...]


def _layer23(z2, y1, dinv2d, c2d, W2, b2, W3, b3, n):
    nb = n // BN
    nfloat = jnp.full((1, 1), 1.0 / n, dtype=jnp.float32)
    return pl.pallas_call(
        _k2_body,
        grid=(nb,),
        in_specs=[
            pl.BlockSpec((BN, 128), lambda i: (i, 0)),
            pl.BlockSpec((BN, 128), lambda i: (i, 0)),
            pl.BlockSpec((BN, 1), lambda i: (i, 0)),
            pl.BlockSpec((BN, 1), lambda i: (i, 0)),
            pl.BlockSpec((128, 256), lambda i: (0, 0)),
            pl.BlockSpec((1, 256), lambda i: (0, 0)),
            pl.BlockSpec((256, 512), lambda i: (0, 0)),
            pl.BlockSpec((1, 512), lambda i: (0, 0)),
            pl.BlockSpec((1, 1), lambda i: (0, 0)),
        ],
        out_specs=pl.BlockSpec((1, 512), lambda i: (0, 0)),
        out_shape=jax.ShapeDtypeStruct((1, 512), jnp.float32),
        scratch_shapes=[pltpu.VMEM((1, 256), jnp.float32)],
    )(z2, y1, dinv2d, c2d, W2, b2.reshape(1, 256), W3, b3.reshape(1, 512),
      nfloat)


def kernel(atom_types, bond_types, edge_index, atom_table, bond_table,
           W1, b1, W2, b2, W3, b3):
    src, dst = edge_index[0], edge_index[1]
    n = atom_types.shape[0]
    e = src.shape[0]
    ewtab = bond_table.mean(axis=-1)
    epad = -(-e // (NS * EB)) * (NS * EB)
    srcp = jnp.concatenate([src, jnp.zeros((epad - e,), src.dtype)])
    dstp = jnp.concatenate([dst, jnp.zeros((epad - e,), dst.dtype)])
    src2d = srcp.reshape(epad // IW, IW)
    dst2d = dstp.reshape(epad // IW, IW)
    btp = jnp.concatenate([bond_types,
                           jnp.full((epad - e,), 10, bond_types.dtype)])
    bt2d = btp.reshape(epad // IW, IW)
    ewtab16 = jnp.concatenate([ewtab, jnp.zeros((6,), jnp.float32)])
    zeros16 = jnp.zeros((n, SW), dtype=jnp.float32)
    zeros1 = jnp.zeros((n,), dtype=jnp.float32)
    ewp, degp = _sc_deg_ew(bt2d, dst2d, ewtab16, zeros1, n)
    deg = degp[0] + degp[1] + 1.0
    dinv = jnp.where(deg > 0, deg ** -0.5, 0.0)
    dinv2d = dinv[:, None]
    x0 = atom_table[atom_types]
    y0 = dinv2d * x0
    z1 = _sc_aggregate(y0, src2d, dst2d, ewp, zeros16, n, 64)
    y1 = _layer1(z1, y0, dinv2d, W1, b1, n)
    z2 = _sc_aggregate(y1, src2d, dst2d, ewp, zeros16, n, 128)
    up = _sc_u(src2d, dst2d, ewp, dinv, zeros1, n)
    c = dinv * (up[0] + up[1]) + dinv * dinv
    return _layer23(z2, y1, dinv2d, c[:, None], W2, b2, W3, b3, n)


# speedup vs baseline: 24.0807x; 1.4853x over previous
import functools
import jax
import jax.numpy as jnp
from jax import lax
from jax.experimental import pallas as pl
from jax.experimental.pallas import tpu as pltpu
from jax.experimental.pallas import tpu_sc as plsc

BN = 2000

NC, NS = 2, 16
IW = 128
RPB = 8
EB = RPB * IW
SW = 16


def _sc_edge_scatter(y_slices, src2d, dst2d, ew, zeros16, n):
    S = len(y_slices)
    nblk = src2d.shape[0] // RPB
    per_tile = nblk // NS
    chunk = -(-(n // NS) // 8) * 8
    last = n - (NS - 1) * chunk

    mesh = plsc.VectorSubcoreMesh(core_axis_name="c", subcore_axis_name="s")

    @functools.partial(
        pl.kernel, mesh=mesh,
        out_type=[jax.ShapeDtypeStruct((n, SW), jnp.float32)] * S,
        scratch_types=[
            pltpu.VMEM((RPB, IW), jnp.int32),
            pltpu.VMEM((RPB, IW), jnp.int32),
            pltpu.VMEM((RPB, IW), jnp.int32),
            pltpu.VMEM((RPB, IW), jnp.int32),
            pltpu.VMEM((EB,), jnp.float32),
            pltpu.VMEM((EB,), jnp.float32),
            pltpu.VMEM((EB, SW), jnp.float32),
            pltpu.VMEM((EB, SW), jnp.float32),
            pltpu.VMEM_SHARED((n, SW), jnp.float32),
            pltpu.SemaphoreType.DMA,
            pltpu.SemaphoreType.DMA,
            pltpu.SemaphoreType.DMA,
            pltpu.SemaphoreType.DMA,
        ],
        compiler_params=pltpu.CompilerParams(use_tc_tiling_on_sc=False),
    )
    def scat(*refs):
        ys = refs[:S]
        src_h, dst_h, ew_h, zer_h = refs[S:S + 4]
        zs = refs[S + 4:2 * S + 4]
        (srcv0, srcv1, dstv0, dstv1, ewv0, ewv1, rows0, rows1, zsh,
         semi0, semi1, semg0, semg1) = refs[2 * S + 4:]
        srcv = (srcv0, srcv1)
        dstv = (dstv0, dstv1)
        ewv = (ewv0, ewv1)
        rows = (rows0, rows1)
        semi = (semi0, semi1)
        semg = (semg0, semg1)
        cid = lax.axis_index("c")
        sid = lax.axis_index("s")

        def chunk_copy(mk_src, mk_dst):
            @pl.when(sid < NS - 1)
            def _():
                sl = pl.ds(sid * chunk, chunk)
                pltpu.sync_copy(mk_src(sl), mk_dst(sl))

            @pl.when(sid == NS - 1)
            def _():
                sl = pl.ds((NS - 1) * chunk, last)
                pltpu.sync_copy(mk_src(sl), mk_dst(sl))

        def do_slice(y_h, z_h):
            chunk_copy(lambda sl: zer_h.at[sl, :], lambda sl: zsh.at[sl, :])
            plsc.subcore_barrier()

            def load_idx(t, b):
                bid = sid + NS * t
                pltpu.async_copy(src_h.at[pl.ds(bid * RPB, RPB), :],
                                 srcv[b], semi[b])
                pltpu.async_copy(dst_h.at[pl.ds(bid * RPB, RPB), :],
                                 dstv[b], semi[b])
                pltpu.async_copy(ew_h.at[pl.ds(bid * EB, EB)], ewv[b],
                                 semi[b])

            def wait_idx(b):
                pltpu.make_async_copy(src_h.at[pl.ds(0, RPB), :],
                                      srcv[b], semi[b]).wait()
                pltpu.make_async_copy(dst_h.at[pl.ds(0, RPB), :],
                                      dstv[b], semi[b]).wait()
                pltpu.make_async_copy(ew_h.at[pl.ds(0, EB)], ewv[b],
                                      semi[b]).wait()

            def issue_gathers(b):
                for j in range(RPB):
                    pltpu.async_copy(y_h.at[srcv[b].at[j]],
                                     rows[b].at[pl.ds(j * IW, IW), :],
                                     semg[b])

            def wait_gathers(b):
                pltpu.make_async_copy(y_h.at[pl.ds(0, EB), :], rows[b],
                                      semg[b]).wait()

            def scale_scatter(b):
                def scale_body(g, _):
                    wv = ewv[b][pl.ds(g * 16, 16)]
                    for l in range(16):
                        e = g * 16 + l
                        rows[b][e, pl.ds(0, SW)] = (
                            rows[b][e, pl.ds(0, SW)] * wv[l])
                    return 0

                lax.fori_loop(0, EB // 16, scale_body, 0)
                for j in range(RPB):
                    pltpu.sync_copy(rows[b].at[pl.ds(j * IW, IW), :],
                                    zsh.at[dstv[b].at[j]], add=True)

            T = per_tile
            pairs = (T - 1) // 2
            load_idx(0, 0)
            wait_idx(0)
            issue_gathers(0)

            def pair_body(k, _):
                t0 = 2 * k
                load_idx(t0 + 1, 1)
                wait_idx(1)
                issue_gathers(1)
                wait_gathers(0)
                scale_scatter(0)
                load_idx(t0 + 2, 0)
                wait_idx(0)
                issue_gathers(0)
                wait_gathers(1)
                scale_scatter(1)
                return 0

            lax.fori_loop(0, pairs, pair_body, 0)
            for ti, t in enumerate(range(2 * pairs, T)):
                b = ti % 2
                if ti > 0:
                    load_idx(t, b)
                    wait_idx(b)
                    issue_gathers(b)
                wait_gathers(b)
                scale_scatter(b)
            plsc.subcore_barrier()
            chunk_copy(lambda sl: zsh.at[sl, :], lambda sl: z_h.at[sl, :])
            plsc.subcore_barrier()

        for kk in range(S // 2):
            for cc in range(NC):
                @pl.when(cid == cc)
                def _(kk=kk, cc=cc):
                    do_slice(ys[NC * kk + cc], zs[NC * kk + cc])

    return scat(*y_slices, src2d, dst2d, ew, zeros16)


def _sc_deg_ew(bt2d, dst2d, ewtab16, zeros1, n):
    epad = bt2d.shape[0] * IW
    nblk = bt2d.shape[0] // RPB
    per_tile = -(-nblk // (NS * NC))
    chunk = -(-(n // NS) // 8) * 8
    last = n - (NS - 1) * chunk

    mesh = plsc.VectorSubcoreMesh(core_axis_name="c", subcore_axis_name="s")

    @functools.partial(
        pl.kernel, mesh=mesh,
        out_type=[jax.ShapeDtypeStruct((epad,), jnp.float32),
                  jax.ShapeDtypeStruct((NC, n), jnp.float32)],
        scratch_types=[
            pltpu.VMEM((RPB, IW), jnp.int32),
            pltpu.VMEM((RPB, IW), jnp.int32),
            pltpu.VMEM((EB,), jnp.float32),
            pltpu.VMEM((16,), jnp.float32),
            pltpu.VMEM_SHARED((n,), jnp.float32),
            pltpu.SemaphoreType.DMA,
        ],
        compiler_params=pltpu.CompilerParams(use_tc_tiling_on_sc=False,
                                             needs_layout_passes=False),
    )
    def degk(bt_h, dst_h, tab_h, zer_h, ew_h, deg_h, btv, dstv, ewv, tabv,
             dsh, sem):
        cid = lax.axis_index("c")
        sid = lax.axis_index("s")
        wid = sid * NC + cid
        pltpu.sync_copy(tab_h, tabv)

        @pl.when(sid < NS - 1)
        def _():
            pltpu.sync_copy(zer_h.at[pl.ds(sid * chunk, chunk)],
                            dsh.at[pl.ds(sid * chunk, chunk)])

        @pl.when(sid == NS - 1)
        def _():
            pltpu.sync_copy(zer_h.at[pl.ds((NS - 1) * chunk, last)],
                            dsh.at[pl.ds((NS - 1) * chunk, last)])

        plsc.subcore_barrier()

        def block_body(i, _):
            bid = wid + NS * NC * i

            @pl.when(bid < nblk)
            def _():
                pltpu.sync_copy(bt_h.at[pl.ds(bid * RPB, RPB), :], btv)
                pltpu.sync_copy(dst_h.at[pl.ds(bid * RPB, RPB), :], dstv)

                def gat(g, _):
                    j, k = g // (IW // 16), g % (IW // 16)
                    bt16 = btv[j, pl.ds(k * 16, 16)]
                    ewv[pl.ds(g * 16, 16)] = plsc.load_gather(tabv, [bt16])
                    return 0

                lax.fori_loop(0, EB // 16, gat, 0)
                pltpu.sync_copy(ewv, ew_h.at[pl.ds(bid * EB, EB)])
                for j in range(RPB):
                    pltpu.sync_copy(ewv.at[pl.ds(j * IW, IW)],
                                    dsh.at[dstv.at[j]], add=True)
            return 0

        lax.fori_loop(0, per_tile, block_body, 0)
        plsc.subcore_barrier()

        @pl.when(sid == 0)
        def _():
            pltpu.sync_copy(dsh, deg_h.at[cid])

    return degk(bt2d, dst2d, ewtab16, zeros1)


def _sc_u(src2d, dst2d, ew, dinv, zeros1, n):
    nblk = src2d.shape[0] // RPB
    per_tile = -(-nblk // (NS * NC))
    chunk = -(-(n // NS) // 8) * 8
    last = n - (NS - 1) * chunk

    mesh = plsc.VectorSubcoreMesh(core_axis_name="c", subcore_axis_name="s")

    @functools.partial(
        pl.kernel, mesh=mesh,
        out_type=jax.ShapeDtypeStruct((NC, n), jnp.float32),
        scratch_types=[
            pltpu.VMEM((RPB, IW), jnp.int32),
            pltpu.VMEM((RPB, IW), jnp.int32),
            pltpu.VMEM((EB,), jnp.float32),
            pltpu.VMEM((EB,), jnp.float32),
            pltpu.VMEM((n,), jnp.float32),
            pltpu.VMEM_SHARED((n,), jnp.float32),
            pltpu.SemaphoreType.DMA,
        ],
        compiler_params=pltpu.CompilerParams(use_tc_tiling_on_sc=False,
                                             needs_layout_passes=False),
    )
    def uk(src_h, dst_h, ew_h, dinv_h, zer_h, u_h, srcv, dstv, ewv, valv,
           dinvv, ush, sem):
        cid = lax.axis_index("c")
        sid = lax.axis_index("s")
        wid = sid * NC + cid
        pltpu.sync_copy(dinv_h, dinvv)

        @pl.when(sid < NS - 1)
        def _():
            pltpu.sync_copy(zer_h.at[pl.ds(sid * chunk, chunk)],
                            ush.at[pl.ds(sid * chunk, chunk)])

        @pl.when(sid == NS - 1)
        def _():
            pltpu.sync_copy(zer_h.at[pl.ds((NS - 1) * chunk, last)],
                            ush.at[pl.ds((NS - 1) * chunk, last)])

        plsc.subcore_barrier()

        def block_body(i, _):
            bid = wid + NS * NC * i

            @pl.when(bid < nblk)
            def _():
                pltpu.sync_copy(src_h.at[pl.ds(bid * RPB, RPB), :], srcv)
                pltpu.sync_copy(dst_h.at[pl.ds(bid * RPB, RPB), :], dstv)
                pltpu.sync_copy(ew_h.at[pl.ds(bid * EB, EB)], ewv)

                def gat(g, _):
                    j, k = g // (IW // 16), g % (IW // 16)
                    d16 = dstv[j, pl.ds(k * 16, 16)]
                    dv = plsc.load_gather(dinvv, [d16])
                    valv[pl.ds(g * 16, 16)] = dv * ewv[pl.ds(g * 16, 16)]
                    return 0

                lax.fori_loop(0, EB // 16, gat, 0)
                for j in range(RPB):
                    pltpu.sync_copy(valv.at[pl.ds(j * IW, IW)],
                                    ush.at[srcv.at[j]], add=True)
            return 0

        lax.fori_loop(0, per_tile, block_body, 0)
        plsc.subcore_barrier()

        @pl.when(sid == 0)
        def _():
            pltpu.sync_copy(ush, u_h.at[cid])

    return uk(src2d, dst2d, ew, dinv, zeros1)


def _k0_body(types, dinv, tab, *outs):
    onehot = jnp.where(
        lax.broadcasted_iota(jnp.int32, (BN, 128), 1) == types[...],
        1.0, 0.0)
    x0 = jnp.dot(onehot, tab[...], preferred_element_type=jnp.float32)
    y0 = dinv[...] * x0
    for k in range(4):
        outs[k][...] = y0[:, SW * k:SW * (k + 1)]


def _embed(types2d, dinv2d, tabpad, n):
    nb = n // BN
    return pl.pallas_call(
        _k0_body,
        grid=(nb,),
        in_specs=[
            pl.BlockSpec((BN, 1), lambda i: (i, 0)),
            pl.BlockSpec((BN, 1), lambda i: (i, 0)),
            pl.BlockSpec((128, 64), lambda i: (0, 0)),
        ],
        out_specs=[pl.BlockSpec((BN, SW), lambda i: (i, 0))] * 4,
        out_shape=[jax.ShapeDtypeStruct((n, SW), jnp.float32)] * 4,
    )(types2d, dinv2d, tabpad)


def _k1_body(z1a, z1b, z1c, z1d, y0a, y0b, y0c, y0d, dinv, W1, b1, *y1s):
    z1 = jnp.concatenate([z1a[...], z1b[...], z1c[...], z1d[...]], axis=1)
    y0 = jnp.concatenate([y0a[...], y0b[...], y0c[...], y0d[...]], axis=1)
    a = dinv[...] * (z1 + y0)
    h = jnp.dot(a, W1[...], preferred_element_type=jnp.float32) + b1[...]
    y1 = dinv[...] * jnp.maximum(h, 0.0)
    for k in range(8):
        y1s[k][...] = y1[:, SW * k:SW * (k + 1)]


def _layer1(z1s, y0s, dinv2d, W1, b1, n):
    nb = n // BN
    return pl.pallas_call(
        _k1_body,
        grid=(nb,),
        in_specs=[pl.BlockSpec((BN, SW), lambda i: (i, 0))] * 8 + [
            pl.BlockSpec((BN, 1), lambda i: (i, 0)),
            pl.BlockSpec((64, 128), lambda i: (0, 0)),
            pl.BlockSpec((1, 128), lambda i: (0, 0)),
        ],
        out_specs=[pl.BlockSpec((BN, SW), lambda i: (i, 0))] * 8,
        out_shape=[jax.ShapeDtypeStruct((n, SW), jnp.float32)] * 8,
    )(*z1s, *y0s, dinv2d, W1, b1.reshape(1, 128))


def _k2_body(*args):
    (z2s, y1s, (dinv, c, W2, b2, W3, b3, nfloat, out, racc)) = (
        args[:8], args[8:16], args[16:])
    i = pl.program_id(0)
    z2 = jnp.concatenate([r[...] for r in z2s], axis=1)
    y1 = jnp.concatenate([r[...] for r in y1s], axis=1)
    a = dinv[...] * (z2 + y1)
    h = jnp.dot(a, W2[...], preferred_element_type=jnp.float32) + b2[...]
    x2 = jnp.maximum(h, 0.0)
    r = lax.dot_general(c[...], x2, (((0,), (0,)), ((), ())),
                        preferred_element_type=jnp.float32)

    @pl.when(i == 0)
    def _():
        racc[...] = r

    @pl.when(i > 0)
    def _():
        racc[...] = racc[...] + r

    @pl.when(i == pl.num_programs(0) - 1)
    def _():
        out[...] = jnp.dot(racc[...] * nfloat[0, 0], W3[...],
                           preferred_element_type=jnp.float32) + b3[...]


def _layer23(z2s, y1s, dinv2d, c2d, W2, b2, W3, b3, n):
    nb = n // BN
    nfloat = jnp.full((1, 1), 1.0 / n, dtype=jnp.float32)
    return pl.pallas_call(
        _k2_body,
        grid=(nb,),
        in_specs=[pl.BlockSpec((BN, SW), lambda i: (i, 0))] * 16 + [
            pl.BlockSpec((BN, 1), lambda i: (i, 0)),
            pl.BlockSpec((BN, 1), lambda i: (i, 0)),
            pl.BlockSpec((128, 256), lambda i: (0, 0)),
            pl.BlockSpec((1, 256), lambda i: (0, 0)),
            pl.BlockSpec((256, 512), lambda i: (0, 0)),
            pl.BlockSpec((1, 512), lambda i: (0, 0)),
            pl.BlockSpec((1, 1), lambda i: (0, 0)),
        ],
        out_specs=pl.BlockSpec((1, 512), lambda i: (0, 0)),
        out_shape=jax.ShapeDtypeStruct((1, 512), jnp.float32),
        scratch_shapes=[pltpu.VMEM((1, 256), jnp.float32)],
    )(*z2s, *y1s, dinv2d, c2d, W2, b2.reshape(1, 256), W3,
      b3.reshape(1, 512), nfloat)


def kernel(atom_types, bond_types, edge_index, atom_table, bond_table,
           W1, b1, W2, b2, W3, b3):
    src, dst = edge_index[0], edge_index[1]
    n = atom_types.shape[0]
    e = src.shape[0]
    ewtab = bond_table.mean(axis=-1)
    epad = -(-e // (NS * EB)) * (NS * EB)
    srcp = jnp.concatenate([src, jnp.zeros((epad - e,), src.dtype)])
    dstp = jnp.concatenate([dst, jnp.zeros((epad - e,), dst.dtype)])
    src2d = srcp.reshape(epad // IW, IW)
    dst2d = dstp.reshape(epad // IW, IW)
    btp = jnp.concatenate([bond_types,
                           jnp.full((epad - e,), 10, bond_types.dtype)])
    bt2d = btp.reshape(epad // IW, IW)
    ewtab16 = jnp.concatenate([ewtab, jnp.zeros((6,), jnp.float32)])
    zeros16 = jnp.zeros((n, SW), dtype=jnp.float32)
    zeros1 = jnp.zeros((n,), dtype=jnp.float32)
    ewp, degp = _sc_deg_ew(bt2d, dst2d, ewtab16, zeros1, n)
    deg = degp[0] + degp[1] + 1.0
    dinv = jnp.where(deg > 0, deg ** -0.5, 0.0)
    dinv2d = dinv[:, None]
    tabpad = jnp.zeros((128, 64), jnp.float32).at[:100].set(atom_table)
    y0s = _embed(atom_types[:, None], dinv2d, tabpad, n)
    z1s = _sc_edge_scatter(tuple(y0s), src2d, dst2d, ewp, zeros16, n)
    y1s = _layer1(z1s, y0s, dinv2d, W1, b1, n)
    z2s = _sc_edge_scatter(tuple(y1s), src2d, dst2d, ewp, zeros16, n)
    up = _sc_u(src2d, dst2d, ewp, dinv, zeros1, n)
    c = dinv * (up[0] + up[1]) + dinv * dinv
    return _layer23(z2s, y1s, dinv2d, c[:, None], W2, b2, W3, b3, n)
